# baseline (speedup 1.0000x reference)
import jax
import jax.numpy as jnp
import numpy as np
from jax.experimental import pallas as pl

F32 = jnp.float32
BF16 = jnp.bfloat16
NEG = -1e30


def _mask_valid(H, W, P):
    return ((np.arange(H * P) % P) < W).astype(np.float32)[:, None]


_MV0 = _mask_valid(84, 84, 96)
_MV1 = _mask_valid(42, 42, 48)
_MV2 = _mask_valid(21, 21, 24)
_MV3 = _mask_valid(11, 11, 12)

_SEL_NP = np.zeros((128, 11 * 12), np.float32)
for _t in range(121):
    _SEL_NP[_t, (_t // 11) * 12 + (_t % 11)] = 1.0
_SEL = _SEL_NP

_RM = (np.arange(128) >= 121).astype(np.float32)[:, None] * np.float32(NEG)

_EYE18 = np.eye(18, dtype=np.float32)

_S0 = (84, 96, 104)
_S1 = (42, 48, 56)
_S2 = (21, 24, 32)
_S3 = (11, 12, 24)


def _conv(zb, w_ref, b_ref, H, P, D):
    L = H * P
    w = w_ref[...]
    b = b_ref[...]
    base = D - P - 1
    zh = zb.astype(BF16)
    acc = None
    if P % 8 == 0:
        L2 = L + 2 * P
        for dx in range(3):
            zdx = zh[base + dx: base + dx + L2, :]
            for dy in range(3):
                t = jnp.dot(zdx[dy * P: dy * P + L, :], w[dy * 3 + dx],
                            preferred_element_type=F32)
                acc = t if acc is None else acc + t
    else:
        for dy in range(3):
            for dx in range(3):
                s0 = base + dy * P + dx
                t = jnp.dot(zh[s0: s0 + L, :], w[dy * 3 + dx],
                            preferred_element_type=F32)
                acc = t if acc is None else acc + t
    return acc + b


def _embed(y, mv, D, C):
    zc = jnp.zeros((D, C), F32)
    return jnp.concatenate([zc, y * mv, zc], axis=0)


def _resblock(zb, wa, ba, wb, bb, mv, H, P, D, C):
    r = jnp.maximum(zb, 0.0)
    h = _conv(r, wa, ba, H, P, D)
    h = jnp.maximum(h, 0.0)
    h2 = _conv(_embed(h, mv, D, C), wb, bb, H, P, D)
    return zb + _embed(h2, mv, D, C)


def _pool_embed(y, mv, H, P, pad_lo, Ho, mvn, Dn, C):
    Hm = H + pad_lo
    Lr = (Hm + 2) * P
    ym = jnp.where(mv > 0, y, NEG)
    parts = []
    if pad_lo:
        parts.append(jnp.full((pad_lo * P + pad_lo, C), NEG, F32))
    parts.append(ym)
    tail = Lr + 8 - H * P - (pad_lo * P + pad_lo)
    parts.append(jnp.full((tail, C), NEG, F32))
    ze = jnp.concatenate(parts, axis=0)
    r = jnp.maximum(jnp.maximum(ze[0:Lr], ze[1:Lr + 1]), ze[2:Lr + 2])
    r4 = r.reshape((Hm + 2) // 2, 2, P, C)
    E, O = r4[:, 0], r4[:, 1]
    m3 = jnp.maximum(jnp.maximum(E[:Ho], O[:Ho]), E[1:Ho + 1])
    wc = m3.reshape(Ho * P, C).reshape(Ho * P // 2, 2, C)[:, 0, :]
    zc = jnp.zeros((Dn, C), F32)
    return jnp.concatenate([zc, wc * mvn, zc], axis=0)


def _mega(*refs):
    (z0_ref,
     w_s0c, b_s0c, w_s0a, b_s0a, w_s0b, b_s0b, w_s0c2, b_s0c2, w_s0d, b_s0d,
     w_s1c, b_s1c, w_s1a, b_s1a, w_s1b, b_s1b, w_s1c2, b_s1c2, w_s1d, b_s1d,
     w_s2c, b_s2c, w_s2a, b_s2a, w_s2b, b_s2b, w_s2c2, b_s2c2, w_s2d, b_s2d,
     mv0_ref, mv1_ref, mv2_ref, mv3_ref, sel_ref, phi_ref, rm_ref,
     w1_ref, b1_ref, w2_ref, b2_ref, whe_ref, eye_ref, bh_ref,
     out_ref) = refs
    mv0, mv1 = mv0_ref[...], mv1_ref[...]
    mv2, mv3 = mv2_ref[...], mv3_ref[...]

    y = jnp.dot(z0_ref[...], w_s0c[...],
                preferred_element_type=F32) + b_s0c[...]
    zb = _pool_embed(y, mv0, 84, 96, 0, 42, mv1, _S1[2], 64)

    H, P, D = _S1
    zb = _resblock(zb, w_s0a, b_s0a, w_s0b, b_s0b, mv1, H, P, D, 64)
    zb = _resblock(zb, w_s0c2, b_s0c2, w_s0d, b_s0d, mv1, H, P, D, 64)
    y = _conv(zb, w_s1c, b_s1c, H, P, D)
    zb = _pool_embed(y, mv1, H, P, 0, 21, mv2, _S2[2], 128)

    H, P, D = _S2
    zb = _resblock(zb, w_s1a, b_s1a, w_s1b, b_s1b, mv2, H, P, D, 128)
    zb = _resblock(zb, w_s1c2, b_s1c2, w_s1d, b_s1d, mv2, H, P, D, 128)
    y = _conv(zb, w_s2c, b_s2c, H, P, D)
    zb = _pool_embed(y, mv2, H, P, 1, 11, mv3, _S3[2], 128)

    H, P, D = _S3
    zb = _resblock(zb, w_s2a, b_s2a, w_s2b, b_s2b, mv3, H, P, D, 128)
    zb = _resblock(zb, w_s2c2, b_s2c2, w_s2d, b_s2d, mv3, H, P, D, 128)

    xe = jnp.maximum(zb[24:24 + 132, :], 0.0)
    tokens = jnp.dot(sel_ref[...], xe, preferred_element_type=F32)
    logits = jnp.dot(tokens, phi_ref[...], preferred_element_type=F32)
    lm = logits + rm_ref[...]
    lm = lm - jnp.max(lm, axis=0, keepdims=True)
    el = jnp.exp(lm)
    disp = el / jnp.sum(el, axis=0, keepdims=True)
    slots = jax.lax.dot_general(disp, tokens, (((0,), (0,)), ((), ())),
                                preferred_element_type=F32)
    ys = []
    for e in range(8):
        se = slots[15 * e:15 * e + 15, :].astype(BF16)
        h = jnp.maximum(jnp.dot(se, w1_ref[e], preferred_element_type=F32)
                        + b1_ref[e], 0.0)
        ys.append(jnp.dot(h.astype(BF16), w2_ref[e],
                          preferred_element_type=F32) + b2_ref[e])
    yall = jnp.concatenate(ys, axis=0)
    cl = logits - jnp.max(logits, axis=1, keepdims=True)
    ec = jnp.exp(cl)
    comb = ec / jnp.sum(ec, axis=1, keepdims=True)
    out = jnp.dot(comb, yall, preferred_element_type=F32)
    eye = eye_ref[...]
    q = bh_ref[...]
    for k in range(18):
        q = q + jnp.sum(out * whe_ref[k]) * eye[k:k + 1, :]
    out_ref[...] = q


def kernel(x, key, params):
    del key
    p = params
    s0, s1, s2 = p['stage0'], p['stage1'], p['stage2']

    def w9(w):
        return w.reshape(9, w.shape[2], w.shape[3]).astype(BF16)

    def b2d(b):
        return b.reshape(1, -1)

    xpad = jnp.pad(x, ((1, 1), (1, 13), (0, 0)))
    z0 = jnp.concatenate(
        [xpad[dy:dy + 84, dx:dx + 96].reshape(84 * 96, 4)
         for dy in range(3) for dx in range(3)], axis=1).astype(BF16)
    phi2 = p['phi'].reshape(128, 120)
    whe = jnp.pad(p['W_head'].reshape(121, 128, 18),
                  ((0, 7), (0, 0), (0, 0))).transpose(2, 0, 1)

    def stage_args(sp):
        cw = sp['conv_w']
        cw = (cw.reshape(36, 64).astype(BF16) if cw.shape[2] == 4
              else w9(cw))
        return (cw, b2d(sp['conv_b']),
                w9(sp['b0_c0_w']), b2d(sp['b0_c0_b']),
                w9(sp['b0_c1_w']), b2d(sp['b0_c1_b']),
                w9(sp['b1_c0_w']), b2d(sp['b1_c0_b']),
                w9(sp['b1_c1_w']), b2d(sp['b1_c1_b']))

    q = pl.pallas_call(
        _mega,
        out_shape=jax.ShapeDtypeStruct((1, 18), F32),
    )(z0, *stage_args(s0), *stage_args(s1), *stage_args(s2),
      _MV0, _MV1, _MV2, _MV3, _SEL, phi2, _RM,
      p['W1'].astype(BF16), p['b1'].reshape(8, 1, 512),
      p['W2'].astype(BF16), p['b2'].reshape(8, 1, 128),
      whe.astype(BF16), _EYE18, p['b_head'].reshape(1, 18))
    return q.reshape(18)


# speedup vs baseline: 1.2145x; 1.2145x over previous
import jax
import jax.numpy as jnp
import numpy as np
from jax.experimental import pallas as pl

F32 = jnp.float32
BF16 = jnp.bfloat16
NEG = -1e30


def _mask_valid(H, W, P):
    return ((np.arange(H * P) % P) < W).astype(np.float32)[:, None]


_MV0 = _mask_valid(84, 84, 96)
_MV1 = _mask_valid(42, 42, 48)
_MV2 = _mask_valid(21, 21, 24)
_MV3 = _mask_valid(11, 11, 12)

_SEL_NP = np.zeros((128, 11 * 12), np.float32)
for _t in range(121):
    _SEL_NP[_t, (_t // 11) * 12 + (_t % 11)] = 1.0
_SEL = _SEL_NP

_RM = (np.arange(128) >= 121).astype(np.float32)[:, None] * np.float32(NEG)

_EYE18 = np.eye(18, dtype=np.float32)

_S0 = (84, 96, 104)
_S1 = (42, 48, 56)
_S2 = (21, 24, 32)
_S3 = (11, 12, 24)


def _conv(zb, w_ref, b_ref, H, P, D):
    L = H * P
    w = w_ref[...].astype(BF16)
    b = b_ref[...]
    base = D - P - 1
    zh = zb.astype(BF16)
    acc = None
    if P % 8 == 0:
        L2 = L + 2 * P
        for dx in range(3):
            zdx = zh[base + dx: base + dx + L2, :]
            for dy in range(3):
                t = jnp.dot(zdx[dy * P: dy * P + L, :], w[dy * 3 + dx],
                            preferred_element_type=F32)
                acc = t if acc is None else acc + t
    else:
        for dy in range(3):
            for dx in range(3):
                s0 = base + dy * P + dx
                t = jnp.dot(zh[s0: s0 + L, :], w[dy * 3 + dx],
                            preferred_element_type=F32)
                acc = t if acc is None else acc + t
    return acc + b


def _embed(y, mv, D, C):
    zc = jnp.zeros((D, C), F32)
    return jnp.concatenate([zc, y * mv, zc], axis=0)


def _resblock(zb, wa, ba, wb, bb, mv, H, P, D, C):
    r = jnp.maximum(zb, 0.0)
    h = _conv(r, wa, ba, H, P, D)
    h = jnp.maximum(h, 0.0)
    h2 = _conv(_embed(h, mv, D, C), wb, bb, H, P, D)
    return zb + _embed(h2, mv, D, C)


def _pool_embed(y, mv, H, P, pad_lo, Ho, mvn, Dn, C):
    Hm = H + pad_lo
    Lr = (Hm + 2) * P
    ym = jnp.where(mv > 0, y, NEG)
    parts = []
    if pad_lo:
        parts.append(jnp.full((pad_lo * P + pad_lo, C), NEG, F32))
    parts.append(ym)
    tail = Lr + 8 - H * P - (pad_lo * P + pad_lo)
    parts.append(jnp.full((tail, C), NEG, F32))
    ze = jnp.concatenate(parts, axis=0)
    r = jnp.maximum(jnp.maximum(ze[0:Lr], ze[1:Lr + 1]), ze[2:Lr + 2])
    r4 = r.reshape((Hm + 2) // 2, 2, P, C)
    E, O = r4[:, 0], r4[:, 1]
    m3 = jnp.maximum(jnp.maximum(E[:Ho], O[:Ho]), E[1:Ho + 1])
    wc = m3.reshape(Ho * P, C).reshape(Ho * P // 2, 2, C)[:, 0, :]
    zc = jnp.zeros((Dn, C), F32)
    return jnp.concatenate([zc, wc * mvn, zc], axis=0)


def _mega(*refs):
    (z0_ref,
     w_s0c, b_s0c, w_s0a, b_s0a, w_s0b, b_s0b, w_s0c2, b_s0c2, w_s0d, b_s0d,
     w_s1c, b_s1c, w_s1a, b_s1a, w_s1b, b_s1b, w_s1c2, b_s1c2, w_s1d, b_s1d,
     w_s2c, b_s2c, w_s2a, b_s2a, w_s2b, b_s2b, w_s2c2, b_s2c2, w_s2d, b_s2d,
     mv0_ref, mv1_ref, mv2_ref, mv3_ref, sel_ref, phi_ref, rm_ref,
     w1_ref, b1_ref, w2_ref, b2_ref, whe_ref, eye_ref, bh_ref,
     out_ref) = refs
    mv0, mv1 = mv0_ref[...], mv1_ref[...]
    mv2, mv3 = mv2_ref[...], mv3_ref[...]

    y = jnp.dot(z0_ref[...], w_s0c[...].astype(BF16),
                preferred_element_type=F32) + b_s0c[...]
    zb = _pool_embed(y, mv0, 84, 96, 0, 42, mv1, _S1[2], 64)

    H, P, D = _S1
    zb = _resblock(zb, w_s0a, b_s0a, w_s0b, b_s0b, mv1, H, P, D, 64)
    zb = _resblock(zb, w_s0c2, b_s0c2, w_s0d, b_s0d, mv1, H, P, D, 64)
    y = _conv(zb, w_s1c, b_s1c, H, P, D)
    zb = _pool_embed(y, mv1, H, P, 0, 21, mv2, _S2[2], 128)

    H, P, D = _S2
    zb = _resblock(zb, w_s1a, b_s1a, w_s1b, b_s1b, mv2, H, P, D, 128)
    zb = _resblock(zb, w_s1c2, b_s1c2, w_s1d, b_s1d, mv2, H, P, D, 128)
    y = _conv(zb, w_s2c, b_s2c, H, P, D)
    zb = _pool_embed(y, mv2, H, P, 1, 11, mv3, _S3[2], 128)

    H, P, D = _S3
    zb = _resblock(zb, w_s2a, b_s2a, w_s2b, b_s2b, mv3, H, P, D, 128)
    zb = _resblock(zb, w_s2c2, b_s2c2, w_s2d, b_s2d, mv3, H, P, D, 128)

    xe = jnp.maximum(zb[24:24 + 132, :], 0.0)
    tokens = jnp.dot(sel_ref[...], xe, preferred_element_type=F32)
    logits = jnp.dot(tokens, phi_ref[...], preferred_element_type=F32)
    lm = logits + rm_ref[...]
    lm = lm - jnp.max(lm, axis=0, keepdims=True)
    el = jnp.exp(lm)
    disp = el / jnp.sum(el, axis=0, keepdims=True)
    slots = jax.lax.dot_general(disp, tokens, (((0,), (0,)), ((), ())),
                                preferred_element_type=F32)
    ys = []
    for e in range(8):
        se = slots[15 * e:15 * e + 15, :].astype(BF16)
        h = jnp.maximum(
            jnp.dot(se, w1_ref[e].astype(BF16),
                    preferred_element_type=F32) + b1_ref[e], 0.0)
        ys.append(jnp.dot(h.astype(BF16), w2_ref[e].astype(BF16),
                          preferred_element_type=F32) + b2_ref[e])
    yall = jnp.concatenate(ys, axis=0)
    cl = logits - jnp.max(logits, axis=1, keepdims=True)
    ec = jnp.exp(cl)
    comb = ec / jnp.sum(ec, axis=1, keepdims=True)
    out = jnp.dot(comb, yall, preferred_element_type=F32)
    eye = eye_ref[...]
    q = bh_ref[...]
    for k in range(18):
        q = q + jnp.sum(out * whe_ref[k]) * eye[k:k + 1, :]
    out_ref[...] = q


def kernel(x, key, params):
    del key
    p = params
    s0, s1, s2 = p['stage0'], p['stage1'], p['stage2']

    def w9(w):
        return w.reshape(9, w.shape[2], w.shape[3])

    def b2d(b):
        return b.reshape(1, -1)

    xpad = jnp.pad(x, ((1, 1), (1, 13), (0, 0)))
    z0 = jnp.concatenate(
        [xpad[dy:dy + 84, dx:dx + 96].reshape(84 * 96, 4)
         for dy in range(3) for dx in range(3)], axis=1).astype(BF16)
    phi2 = p['phi'].reshape(128, 120)
    whe = jnp.pad(p['W_head'].reshape(121, 128, 18),
                  ((0, 7), (0, 0), (0, 0))).transpose(2, 0, 1)

    def stage_args(sp):
        cw = sp['conv_w']
        cw = cw.reshape(36, 64) if cw.shape[2] == 4 else w9(cw)
        return (cw, b2d(sp['conv_b']),
                w9(sp['b0_c0_w']), b2d(sp['b0_c0_b']),
                w9(sp['b0_c1_w']), b2d(sp['b0_c1_b']),
                w9(sp['b1_c0_w']), b2d(sp['b1_c0_b']),
                w9(sp['b1_c1_w']), b2d(sp['b1_c1_b']))

    q = pl.pallas_call(
        _mega,
        out_shape=jax.ShapeDtypeStruct((1, 18), F32),
    )(z0, *stage_args(s0), *stage_args(s1), *stage_args(s2),
      _MV0, _MV1, _MV2, _MV3, _SEL, phi2, _RM,
      p['W1'], p['b1'].reshape(8, 1, 512),
      p['W2'], p['b2'].reshape(8, 1, 128),
      whe, _EYE18, p['b_head'].reshape(1, 18))
    return q.reshape(18)


# speedup vs baseline: 1.2564x; 1.0345x over previous
import jax
import jax.numpy as jnp
import numpy as np
from jax.experimental import pallas as pl
from jax.experimental.pallas import tpu as pltpu

F32 = jnp.float32
BF16 = jnp.bfloat16
NEG = -1e30


def _mask_valid(H, W, P):
    return ((np.arange(H * P) % P) < W).astype(np.float32)[:, None]


_MV0 = _mask_valid(84, 84, 96)
_MV1 = _mask_valid(42, 42, 48)
_MV2 = _mask_valid(21, 21, 24)
_MV3 = _mask_valid(11, 11, 12)

_SEL_NP = np.zeros((128, 11 * 12), np.float32)
for _t in range(121):
    _SEL_NP[_t, (_t // 11) * 12 + (_t % 11)] = 1.0
_SEL = _SEL_NP

_RM = (np.arange(128) >= 121).astype(np.float32)[:, None] * np.float32(NEG)

_EYE18 = np.eye(18, dtype=np.float32)

_S0 = (84, 96, 104)
_S1 = (42, 48, 56)
_S2 = (21, 24, 32)
_S3 = (11, 12, 24)


def _conv(zb, w_ref, b_ref, H, P, D):
    L = H * P
    w = w_ref[...].astype(BF16)
    b = b_ref[...]
    base = D - P - 1
    zh = zb.astype(BF16)
    acc = None
    if P % 8 == 0:
        L2 = L + 2 * P
        for dx in range(3):
            zdx = zh[base + dx: base + dx + L2, :]
            for dy in range(3):
                t = jnp.dot(zdx[dy * P: dy * P + L, :], w[dy * 3 + dx],
                            preferred_element_type=F32)
                acc = t if acc is None else acc + t
    else:
        for dy in range(3):
            for dx in range(3):
                s0 = base + dy * P + dx
                t = jnp.dot(zh[s0: s0 + L, :], w[dy * 3 + dx],
                            preferred_element_type=F32)
                acc = t if acc is None else acc + t
    return acc + b


def _embed(y, mv, D, C):
    zc = jnp.zeros((D, C), F32)
    return jnp.concatenate([zc, y * mv, zc], axis=0)


def _resblock(zb, wa, ba, wb, bb, mv, H, P, D, C):
    r = jnp.maximum(zb, 0.0)
    h = _conv(r, wa, ba, H, P, D)
    h = jnp.maximum(h, 0.0)
    h2 = _conv(_embed(h, mv, D, C), wb, bb, H, P, D)
    return zb + _embed(h2, mv, D, C)


def _pool_embed(y, mv, H, P, pad_lo, Ho, mvn, Dn, C):
    Hm = H + pad_lo
    Lr = (Hm + 2) * P
    ym = jnp.where(mv > 0, y, NEG)
    parts = []
    if pad_lo:
        parts.append(jnp.full((pad_lo * P + pad_lo, C), NEG, F32))
    parts.append(ym)
    tail = Lr + 8 - H * P - (pad_lo * P + pad_lo)
    parts.append(jnp.full((tail, C), NEG, F32))
    ze = jnp.concatenate(parts, axis=0)
    r = jnp.maximum(jnp.maximum(ze[0:Lr], ze[1:Lr + 1]), ze[2:Lr + 2])
    r4 = r.reshape((Hm + 2) // 2, 2, P, C)
    E, O = r4[:, 0], r4[:, 1]
    m3 = jnp.maximum(jnp.maximum(E[:Ho], O[:Ho]), E[1:Ho + 1])
    wc = m3.reshape(Ho * P, C).reshape(Ho * P // 2, 2, C)[:, 0, :]
    zc = jnp.zeros((Dn, C), F32)
    return jnp.concatenate([zc, wc * mvn, zc], axis=0)


def _mega(*refs):
    (z0_ref,
     w_s0c, b_s0c, w_s0a, b_s0a, w_s0b, b_s0b, w_s0c2, b_s0c2, w_s0d, b_s0d,
     w_s1c, b_s1c, w_s1a, b_s1a, w_s1b, b_s1b, w_s1c2, b_s1c2, w_s1d, b_s1d,
     w_s2c, b_s2c, w_s2a, b_s2a, w_s2b, b_s2b, w_s2c2, b_s2c2, w_s2d, b_s2d,
     mv0_ref, mv1_ref, mv2_ref, mv3_ref, sel_ref, phi_ref, rm_ref,
     w1_ref, b1_ref, w2_ref, b2_ref, whe_ref, eye_ref, bh_ref,
     out_ref, w1v, w2v, whev, sem1, sem2, sem3) = refs
    c1 = pltpu.make_async_copy(w1_ref, w1v, sem1)
    c2 = pltpu.make_async_copy(w2_ref, w2v, sem2)
    c3 = pltpu.make_async_copy(whe_ref, whev, sem3)
    c1.start()
    c2.start()
    c3.start()
    mv0, mv1 = mv0_ref[...], mv1_ref[...]
    mv2, mv3 = mv2_ref[...], mv3_ref[...]

    y = jnp.dot(z0_ref[...], w_s0c[...].astype(BF16),
                preferred_element_type=F32) + b_s0c[...]
    zb = _pool_embed(y, mv0, 84, 96, 0, 42, mv1, _S1[2], 64)

    H, P, D = _S1
    zb = _resblock(zb, w_s0a, b_s0a, w_s0b, b_s0b, mv1, H, P, D, 64)
    zb = _resblock(zb, w_s0c2, b_s0c2, w_s0d, b_s0d, mv1, H, P, D, 64)
    y = _conv(zb, w_s1c, b_s1c, H, P, D)
    zb = _pool_embed(y, mv1, H, P, 0, 21, mv2, _S2[2], 128)

    H, P, D = _S2
    zb = _resblock(zb, w_s1a, b_s1a, w_s1b, b_s1b, mv2, H, P, D, 128)
    zb = _resblock(zb, w_s1c2, b_s1c2, w_s1d, b_s1d, mv2, H, P, D, 128)
    y = _conv(zb, w_s2c, b_s2c, H, P, D)
    zb = _pool_embed(y, mv2, H, P, 1, 11, mv3, _S3[2], 128)

    H, P, D = _S3
    zb = _resblock(zb, w_s2a, b_s2a, w_s2b, b_s2b, mv3, H, P, D, 128)
    zb = _resblock(zb, w_s2c2, b_s2c2, w_s2d, b_s2d, mv3, H, P, D, 128)

    xe = jnp.maximum(zb[24:24 + 132, :], 0.0)
    tokens = jnp.dot(sel_ref[...], xe, preferred_element_type=F32)
    logits = jnp.dot(tokens, phi_ref[...], preferred_element_type=F32)
    lm = logits + rm_ref[...]
    lm = lm - jnp.max(lm, axis=0, keepdims=True)
    el = jnp.exp(lm)
    disp = el / jnp.sum(el, axis=0, keepdims=True)
    slots = jax.lax.dot_general(disp, tokens, (((0,), (0,)), ((), ())),
                                preferred_element_type=F32)
    c1.wait()
    c2.wait()
    ys = []
    for e in range(8):
        se = slots[15 * e:15 * e + 15, :].astype(BF16)
        h = jnp.maximum(
            jnp.dot(se, w1v[e].astype(BF16),
                    preferred_element_type=F32) + b1_ref[e], 0.0)
        ys.append(jnp.dot(h.astype(BF16), w2v[e].astype(BF16),
                          preferred_element_type=F32) + b2_ref[e])
    yall = jnp.concatenate(ys, axis=0)
    cl = logits - jnp.max(logits, axis=1, keepdims=True)
    ec = jnp.exp(cl)
    comb = ec / jnp.sum(ec, axis=1, keepdims=True)
    out = jnp.dot(comb, yall, preferred_element_type=F32)
    c3.wait()
    eye = eye_ref[...]
    q = bh_ref[...]
    for k in range(18):
        q = q + jnp.sum(out * whev[k]) * eye[k:k + 1, :]
    out_ref[...] = q


def kernel(x, key, params):
    del key
    p = params
    s0, s1, s2 = p['stage0'], p['stage1'], p['stage2']

    def w9(w):
        return w.reshape(9, w.shape[2], w.shape[3])

    def b2d(b):
        return b.reshape(1, -1)

    xpad = jnp.pad(x, ((1, 1), (1, 13), (0, 0)))
    z0 = jnp.concatenate(
        [xpad[dy:dy + 84, dx:dx + 96].reshape(84 * 96, 4)
         for dy in range(3) for dx in range(3)], axis=1).astype(BF16)
    phi2 = p['phi'].reshape(128, 120)
    whe = jnp.pad(p['W_head'].reshape(121, 128, 18),
                  ((0, 7), (0, 0), (0, 0))).transpose(2, 0, 1)

    def stage_args(sp):
        cw = sp['conv_w']
        cw = cw.reshape(36, 64) if cw.shape[2] == 4 else w9(cw)
        return (cw, b2d(sp['conv_b']),
                w9(sp['b0_c0_w']), b2d(sp['b0_c0_b']),
                w9(sp['b0_c1_w']), b2d(sp['b0_c1_b']),
                w9(sp['b1_c0_w']), b2d(sp['b1_c0_b']),
                w9(sp['b1_c1_w']), b2d(sp['b1_c1_b']))

    vm = pl.BlockSpec(memory_space=pltpu.MemorySpace.VMEM)
    hb = pl.BlockSpec(memory_space=pltpu.MemorySpace.HBM)
    q = pl.pallas_call(
        _mega,
        out_shape=jax.ShapeDtypeStruct((1, 18), F32),
        in_specs=[vm] * 38 + [hb, vm, hb, vm, hb, vm, vm],
        scratch_shapes=[
            pltpu.VMEM((8, 128, 512), F32),
            pltpu.VMEM((8, 512, 128), F32),
            pltpu.VMEM((18, 128, 128), F32),
            pltpu.SemaphoreType.DMA,
            pltpu.SemaphoreType.DMA,
            pltpu.SemaphoreType.DMA,
        ],
    )(z0, *stage_args(s0), *stage_args(s1), *stage_args(s2),
      _MV0, _MV1, _MV2, _MV3, _SEL, phi2, _RM,
      p['W1'], p['b1'].reshape(8, 1, 512),
      p['W2'], p['b2'].reshape(8, 1, 128),
      whe, _EYE18, p['b_head'].reshape(1, 18))
    return q.reshape(18)


# speedup vs baseline: 1.2942x; 1.0301x over previous
import jax
import jax.numpy as jnp
import numpy as np
from jax.experimental import pallas as pl
from jax.experimental.pallas import tpu as pltpu

F32 = jnp.float32
BF16 = jnp.bfloat16
NEG = -1e30


def _mask_valid(H, W, P):
    return ((np.arange(H * P) % P) < W).astype(np.float32)[:, None]


_MV0 = _mask_valid(84, 84, 96)
_MV1 = _mask_valid(42, 42, 48)
_MV2 = _mask_valid(21, 21, 24)
_MV3 = _mask_valid(11, 11, 12)

_SEL_NP = np.zeros((128, 11 * 12), np.float32)
for _t in range(121):
    _SEL_NP[_t, (_t // 11) * 12 + (_t % 11)] = 1.0
_SEL = _SEL_NP

_RM = (np.arange(128) >= 121).astype(np.float32)[:, None] * np.float32(NEG)

_EYE18 = np.eye(18, dtype=np.float32)

_S0 = (84, 96, 104)
_S1 = (42, 48, 56)
_S2 = (21, 24, 32)
_S3 = (11, 12, 24)


def _conv(zb, w_ref, b_ref, H, P, D):
    L = H * P
    w = w_ref[...].astype(BF16)
    b = b_ref[...]
    base = D - P - 1
    zh = zb.astype(BF16)
    acc = None
    if P % 8 == 0:
        L2 = L + 2 * P
        for dx in range(3):
            zdx = zh[base + dx: base + dx + L2, :]
            for dy in range(3):
                t = jnp.dot(zdx[dy * P: dy * P + L, :], w[dy * 3 + dx],
                            preferred_element_type=F32)
                acc = t if acc is None else acc + t
    else:
        for dy in range(3):
            for dx in range(3):
                s0 = base + dy * P + dx
                t = jnp.dot(zh[s0: s0 + L, :], w[dy * 3 + dx],
                            preferred_element_type=F32)
                acc = t if acc is None else acc + t
    return acc + b


def _embed(y, mv, D, C):
    zc = jnp.zeros((D, C), F32)
    return jnp.concatenate([zc, y * mv, zc], axis=0)


def _resblock(zb, wa, ba, wb, bb, mv, H, P, D, C):
    r = jnp.maximum(zb, 0.0)
    h = _conv(r, wa, ba, H, P, D)
    h = jnp.maximum(h, 0.0)
    h2 = _conv(_embed(h, mv, D, C), wb, bb, H, P, D)
    return zb + _embed(h2, mv, D, C)


def _pool_embed(y, mv, H, P, pad_lo, Ho, mvn, Dn, C):
    Hm = H + pad_lo
    Lr = (Hm + 2) * P
    ym = jnp.where(mv > 0, y, NEG)
    parts = []
    if pad_lo:
        parts.append(jnp.full((pad_lo * P + pad_lo, C), NEG, F32))
    parts.append(ym)
    tail = Lr + 8 - H * P - (pad_lo * P + pad_lo)
    parts.append(jnp.full((tail, C), NEG, F32))
    ze = jnp.concatenate(parts, axis=0)
    r = jnp.maximum(jnp.maximum(ze[0:Lr], ze[1:Lr + 1]), ze[2:Lr + 2])
    r4 = r.reshape((Hm + 2) // 2, 2, P, C)
    E, O = r4[:, 0], r4[:, 1]
    m3 = jnp.maximum(jnp.maximum(E[:Ho], O[:Ho]), E[1:Ho + 1])
    wc = m3.reshape(Ho * P, C).reshape(Ho * P // 2, 2, C)[:, 0, :]
    zc = jnp.zeros((Dn, C), F32)
    return jnp.concatenate([zc, wc * mvn, zc], axis=0)


def _mega(*refs):
    (z0_ref,
     w_s0c, b_s0c, w_s0a, b_s0a, w_s0b, b_s0b, w_s0c2, b_s0c2, w_s0d, b_s0d,
     w_s1c, b_s1c, w_s1a, b_s1a, w_s1b, b_s1b, w_s1c2, b_s1c2, w_s1d, b_s1d,
     w_s2c, b_s2c, w_s2a, b_s2a, w_s2b, b_s2b, w_s2c2, b_s2c2, w_s2d, b_s2d,
     mv0_ref, mv1_ref, mv2_ref, mv3_ref, sel_ref, phi_ref, rm_ref,
     w1_ref, b1_ref, w2_ref, b2_ref, whe_ref, eye_ref, bh_ref,
     out_ref,
     s1cv, s1av, s1bv, s1c2v, s1dv, s2cv, s2av, s2bv, s2c2v, s2dv,
     w1v, w2v, whev,
     m1, m2, m3, m4, m5, m6, m7, m8, m9, m10, sem1, sem2, sem3) = refs
    cs = [pltpu.make_async_copy(src, dst, sm) for src, dst, sm in (
        (w_s1c, s1cv, m1), (w_s1a, s1av, m2), (w_s1b, s1bv, m3),
        (w_s1c2, s1c2v, m4), (w_s1d, s1dv, m5),
        (w_s2c, s2cv, m6), (w_s2a, s2av, m7), (w_s2b, s2bv, m8),
        (w_s2c2, s2c2v, m9), (w_s2d, s2dv, m10))]
    c1 = pltpu.make_async_copy(w1_ref, w1v, sem1)
    c2 = pltpu.make_async_copy(w2_ref, w2v, sem2)
    c3 = pltpu.make_async_copy(whe_ref, whev, sem3)
    for c in cs:
        c.start()
    c1.start()
    c2.start()
    c3.start()
    mv0, mv1 = mv0_ref[...], mv1_ref[...]
    mv2, mv3 = mv2_ref[...], mv3_ref[...]

    y = jnp.dot(z0_ref[...], w_s0c[...].astype(BF16),
                preferred_element_type=F32) + b_s0c[...]
    zb = _pool_embed(y, mv0, 84, 96, 0, 42, mv1, _S1[2], 64)

    H, P, D = _S1
    zb = _resblock(zb, w_s0a, b_s0a, w_s0b, b_s0b, mv1, H, P, D, 64)
    zb = _resblock(zb, w_s0c2, b_s0c2, w_s0d, b_s0d, mv1, H, P, D, 64)
    for c in cs[:5]:
        c.wait()
    y = _conv(zb, s1cv, b_s1c, H, P, D)
    zb = _pool_embed(y, mv1, H, P, 0, 21, mv2, _S2[2], 128)

    H, P, D = _S2
    zb = _resblock(zb, s1av, b_s1a, s1bv, b_s1b, mv2, H, P, D, 128)
    zb = _resblock(zb, s1c2v, b_s1c2, s1dv, b_s1d, mv2, H, P, D, 128)
    for c in cs[5:]:
        c.wait()
    y = _conv(zb, s2cv, b_s2c, H, P, D)
    zb = _pool_embed(y, mv2, H, P, 1, 11, mv3, _S3[2], 128)

    H, P, D = _S3
    zb = _resblock(zb, s2av, b_s2a, s2bv, b_s2b, mv3, H, P, D, 128)
    zb = _resblock(zb, s2c2v, b_s2c2, s2dv, b_s2d, mv3, H, P, D, 128)

    xe = jnp.maximum(zb[24:24 + 132, :], 0.0)
    tokens = jnp.dot(sel_ref[...], xe, preferred_element_type=F32)
    logits = jnp.dot(tokens, phi_ref[...], preferred_element_type=F32)
    lm = logits + rm_ref[...]
    lm = lm - jnp.max(lm, axis=0, keepdims=True)
    el = jnp.exp(lm)
    disp = el / jnp.sum(el, axis=0, keepdims=True)
    slots = jax.lax.dot_general(disp, tokens, (((0,), (0,)), ((), ())),
                                preferred_element_type=F32)
    c1.wait()
    c2.wait()
    ys = []
    for e in range(8):
        se = slots[15 * e:15 * e + 15, :].astype(BF16)
        h = jnp.maximum(
            jnp.dot(se, w1v[e].astype(BF16),
                    preferred_element_type=F32) + b1_ref[e], 0.0)
        ys.append(jnp.dot(h.astype(BF16), w2v[e].astype(BF16),
                          preferred_element_type=F32) + b2_ref[e])
    yall = jnp.concatenate(ys, axis=0)
    cl = logits - jnp.max(logits, axis=1, keepdims=True)
    ec = jnp.exp(cl)
    comb = ec / jnp.sum(ec, axis=1, keepdims=True)
    out = jnp.dot(comb, yall, preferred_element_type=F32)
    c3.wait()
    eye = eye_ref[...]
    q = bh_ref[...]
    for k in range(18):
        q = q + jnp.sum(out * whev[k]) * eye[k:k + 1, :]
    out_ref[...] = q


def kernel(x, key, params):
    del key
    p = params
    s0, s1, s2 = p['stage0'], p['stage1'], p['stage2']

    def w9(w):
        return w.reshape(9, w.shape[2], w.shape[3])

    def b2d(b):
        return b.reshape(1, -1)

    xpad = jnp.pad(x, ((1, 1), (1, 13), (0, 0)))
    z0 = jnp.concatenate(
        [xpad[dy:dy + 84, dx:dx + 96].reshape(84 * 96, 4)
         for dy in range(3) for dx in range(3)], axis=1).astype(BF16)
    phi2 = p['phi'].reshape(128, 120)
    whe = jnp.pad(p['W_head'].reshape(121, 128, 18),
                  ((0, 7), (0, 0), (0, 0))).transpose(2, 0, 1)

    def stage_args(sp):
        cw = sp['conv_w']
        cw = cw.reshape(36, 64) if cw.shape[2] == 4 else w9(cw)
        return (cw, b2d(sp['conv_b']),
                w9(sp['b0_c0_w']), b2d(sp['b0_c0_b']),
                w9(sp['b0_c1_w']), b2d(sp['b0_c1_b']),
                w9(sp['b1_c0_w']), b2d(sp['b1_c0_b']),
                w9(sp['b1_c1_w']), b2d(sp['b1_c1_b']))

    vm = pl.BlockSpec(memory_space=pltpu.MemorySpace.VMEM)
    hb = pl.BlockSpec(memory_space=pltpu.MemorySpace.HBM)
    q = pl.pallas_call(
        _mega,
        out_shape=jax.ShapeDtypeStruct((1, 18), F32),
        in_specs=([vm] * 11 + [hb, vm] * 10 + [vm] * 7
                  + [hb, vm, hb, vm, hb, vm, vm]),
        scratch_shapes=[
            pltpu.VMEM((9, 64, 128), F32),
            pltpu.VMEM((9, 128, 128), F32),
            pltpu.VMEM((9, 128, 128), F32),
            pltpu.VMEM((9, 128, 128), F32),
            pltpu.VMEM((9, 128, 128), F32),
            pltpu.VMEM((9, 128, 128), F32),
            pltpu.VMEM((9, 128, 128), F32),
            pltpu.VMEM((9, 128, 128), F32),
            pltpu.VMEM((9, 128, 128), F32),
            pltpu.VMEM((9, 128, 128), F32),
            pltpu.VMEM((8, 128, 512), F32),
            pltpu.VMEM((8, 512, 128), F32),
            pltpu.VMEM((18, 128, 128), F32),
        ] + [pltpu.SemaphoreType.DMA] * 13,
    )(z0, *stage_args(s0), *stage_args(s1), *stage_args(s2),
      _MV0, _MV1, _MV2, _MV3, _SEL, phi2, _RM,
      p['W1'], p['b1'].reshape(8, 1, 512),
      p['W2'], p['b2'].reshape(8, 1, 128),
      whe, _EYE18, p['b_head'].reshape(1, 18))
    return q.reshape(18)


# speedup vs baseline: 1.2972x; 1.0023x over previous
import jax
import jax.numpy as jnp
import numpy as np
from jax.experimental import pallas as pl
from jax.experimental.pallas import tpu as pltpu

F32 = jnp.float32
BF16 = jnp.bfloat16
NEG = -1e30


def _mask_valid(H, W, P):
    return ((np.arange(H * P) % P) < W).astype(np.float32)[:, None]


_MV0 = _mask_valid(84, 84, 96)
_MV1 = _mask_valid(42, 42, 48)
_MV2 = _mask_valid(21, 21, 24)
_MV3 = _mask_valid(11, 11, 12)

_SEL_NP = np.zeros((128, 11 * 12), np.float32)
for _t in range(121):
    _SEL_NP[_t, (_t // 11) * 12 + (_t % 11)] = 1.0
_SEL = _SEL_NP

_RM = (np.arange(128) >= 121).astype(np.float32)[:, None] * np.float32(NEG)

_EYE18 = np.eye(18, dtype=np.float32)

_S0 = (84, 96, 104)
_S1 = (42, 48, 56)
_S2 = (21, 24, 32)
_S3 = (11, 12, 24)


def _conv(zb, w_ref, b_ref, H, P, D):
    L = H * P
    w = w_ref[...].astype(BF16)
    b = b_ref[...]
    base = D - P - 1
    zh = zb.astype(BF16)
    parts = []
    if P % 8 == 0:
        L2 = L + 2 * P
        for dx in range(3):
            zdx = zh[base + dx: base + dx + L2, :]
            for dy in range(3):
                parts.append(jnp.dot(zdx[dy * P: dy * P + L, :],
                                     w[dy * 3 + dx],
                                     preferred_element_type=F32))
    else:
        for dy in range(3):
            for dx in range(3):
                s0 = base + dy * P + dx
                parts.append(jnp.dot(zh[s0: s0 + L, :], w[dy * 3 + dx],
                                     preferred_element_type=F32))
    while len(parts) > 1:
        parts = ([parts[i] + parts[i + 1]
                  for i in range(0, len(parts) - 1, 2)]
                 + ([parts[-1]] if len(parts) % 2 else []))
    return parts[0] + b


def _embed(y, mv, D, C):
    zc = jnp.zeros((D, C), F32)
    return jnp.concatenate([zc, y * mv, zc], axis=0)


def _resblock(zb, wa, ba, wb, bb, mv, H, P, D, C):
    r = jnp.maximum(zb, 0.0)
    h = _conv(r, wa, ba, H, P, D)
    h = jnp.maximum(h, 0.0)
    h2 = _conv(_embed(h, mv, D, C), wb, bb, H, P, D)
    return zb + _embed(h2, mv, D, C)


def _pool_embed(y, mv, H, P, pad_lo, Ho, mvn, Dn, C):
    Hm = H + pad_lo
    Lr = (Hm + 2) * P
    ym = jnp.where(mv > 0, y, NEG)
    parts = []
    if pad_lo:
        parts.append(jnp.full((pad_lo * P + pad_lo, C), NEG, F32))
    parts.append(ym)
    tail = Lr + 8 - H * P - (pad_lo * P + pad_lo)
    parts.append(jnp.full((tail, C), NEG, F32))
    ze = jnp.concatenate(parts, axis=0)
    r = jnp.maximum(jnp.maximum(ze[0:Lr], ze[1:Lr + 1]), ze[2:Lr + 2])
    r4 = r.reshape((Hm + 2) // 2, 2, P, C)
    E, O = r4[:, 0], r4[:, 1]
    m3 = jnp.maximum(jnp.maximum(E[:Ho], O[:Ho]), E[1:Ho + 1])
    wc = m3.reshape(Ho * P, C).reshape(Ho * P // 2, 2, C)[:, 0, :]
    zc = jnp.zeros((Dn, C), F32)
    return jnp.concatenate([zc, wc * mvn, zc], axis=0)


def _mega(*refs):
    (z0_ref,
     w_s0c, b_s0c, w_s0a, b_s0a, w_s0b, b_s0b, w_s0c2, b_s0c2, w_s0d, b_s0d,
     w_s1c, b_s1c, w_s1a, b_s1a, w_s1b, b_s1b, w_s1c2, b_s1c2, w_s1d, b_s1d,
     w_s2c, b_s2c, w_s2a, b_s2a, w_s2b, b_s2b, w_s2c2, b_s2c2, w_s2d, b_s2d,
     mv0_ref, mv1_ref, mv2_ref, mv3_ref, sel_ref, phi_ref, rm_ref,
     w1_ref, b1_ref, w2_ref, b2_ref, whe_ref, eye_ref, bh_ref,
     out_ref,
     s1cv, s1av, s1bv, s1c2v, s1dv, s2cv, s2av, s2bv, s2c2v, s2dv,
     w1v, w2v, whev,
     m1, m2, m3, m4, m5, m6, m7, m8, m9, m10, sem1, sem2, sem3) = refs
    cs = [pltpu.make_async_copy(src, dst, sm) for src, dst, sm in (
        (w_s1c, s1cv, m1), (w_s1a, s1av, m2), (w_s1b, s1bv, m3),
        (w_s1c2, s1c2v, m4), (w_s1d, s1dv, m5),
        (w_s2c, s2cv, m6), (w_s2a, s2av, m7), (w_s2b, s2bv, m8),
        (w_s2c2, s2c2v, m9), (w_s2d, s2dv, m10))]
    c1 = pltpu.make_async_copy(w1_ref, w1v, sem1)
    c2 = pltpu.make_async_copy(w2_ref, w2v, sem2)
    c3 = pltpu.make_async_copy(whe_ref, whev, sem3)
    for c in cs:
        c.start()
    c1.start()
    c2.start()
    c3.start()
    mv0, mv1 = mv0_ref[...], mv1_ref[...]
    mv2, mv3 = mv2_ref[...], mv3_ref[...]

    y = jnp.dot(z0_ref[...], w_s0c[...].astype(BF16),
                preferred_element_type=F32) + b_s0c[...]
    zb = _pool_embed(y, mv0, 84, 96, 0, 42, mv1, _S1[2], 64)

    H, P, D = _S1
    zb = _resblock(zb, w_s0a, b_s0a, w_s0b, b_s0b, mv1, H, P, D, 64)
    zb = _resblock(zb, w_s0c2, b_s0c2, w_s0d, b_s0d, mv1, H, P, D, 64)
    for c in cs[:5]:
        c.wait()
    y = _conv(zb, s1cv, b_s1c, H, P, D)
    zb = _pool_embed(y, mv1, H, P, 0, 21, mv2, _S2[2], 128)

    H, P, D = _S2
    zb = _resblock(zb, s1av, b_s1a, s1bv, b_s1b, mv2, H, P, D, 128)
    zb = _resblock(zb, s1c2v, b_s1c2, s1dv, b_s1d, mv2, H, P, D, 128)
    for c in cs[5:]:
        c.wait()
    y = _conv(zb, s2cv, b_s2c, H, P, D)
    zb = _pool_embed(y, mv2, H, P, 1, 11, mv3, _S3[2], 128)

    H, P, D = _S3
    zb = _resblock(zb, s2av, b_s2a, s2bv, b_s2b, mv3, H, P, D, 128)
    zb = _resblock(zb, s2c2v, b_s2c2, s2dv, b_s2d, mv3, H, P, D, 128)

    xe = jnp.maximum(zb[24:24 + 132, :], 0.0)
    tokens = jnp.dot(sel_ref[...], xe, preferred_element_type=F32)
    logits = jnp.dot(tokens, phi_ref[...], preferred_element_type=F32)
    lm = logits + rm_ref[...]
    lm = lm - jnp.max(lm, axis=0, keepdims=True)
    el = jnp.exp(lm)
    disp = el / jnp.sum(el, axis=0, keepdims=True)
    slots = jax.lax.dot_general(disp, tokens, (((0,), (0,)), ((), ())),
                                preferred_element_type=F32)
    c1.wait()
    c2.wait()
    ys = []
    for e in range(8):
        se = slots[15 * e:15 * e + 15, :].astype(BF16)
        h = jnp.maximum(
            jnp.dot(se, w1v[e].astype(BF16),
                    preferred_element_type=F32) + b1_ref[e], 0.0)
        ys.append(jnp.dot(h.astype(BF16), w2v[e].astype(BF16),
                          preferred_element_type=F32) + b2_ref[e])
    yall = jnp.concatenate(ys, axis=0)
    cl = logits - jnp.max(logits, axis=1, keepdims=True)
    ec = jnp.exp(cl)
    comb = ec / jnp.sum(ec, axis=1, keepdims=True)
    out = jnp.dot(comb, yall, preferred_element_type=F32)
    c3.wait()
    eye = eye_ref[...]
    q = bh_ref[...]
    for k in range(18):
        q = q + jnp.sum(out * whev[k]) * eye[k:k + 1, :]
    out_ref[...] = q


def kernel(x, key, params):
    del key
    p = params
    s0, s1, s2 = p['stage0'], p['stage1'], p['stage2']

    def w9(w):
        return w.reshape(9, w.shape[2], w.shape[3])

    def b2d(b):
        return b.reshape(1, -1)

    xpad = jnp.pad(x, ((1, 1), (1, 13), (0, 0)))
    z0 = jnp.concatenate(
        [xpad[dy:dy + 84, dx:dx + 96].reshape(84 * 96, 4)
         for dy in range(3) for dx in range(3)], axis=1).astype(BF16)
    phi2 = p['phi'].reshape(128, 120)
    whe = jnp.pad(p['W_head'].reshape(121, 128, 18),
                  ((0, 7), (0, 0), (0, 0))).transpose(2, 0, 1)

    def stage_args(sp):
        cw = sp['conv_w']
        cw = cw.reshape(36, 64) if cw.shape[2] == 4 else w9(cw)
        return (cw, b2d(sp['conv_b']),
                w9(sp['b0_c0_w']), b2d(sp['b0_c0_b']),
                w9(sp['b0_c1_w']), b2d(sp['b0_c1_b']),
                w9(sp['b1_c0_w']), b2d(sp['b1_c0_b']),
                w9(sp['b1_c1_w']), b2d(sp['b1_c1_b']))

    vm = pl.BlockSpec(memory_space=pltpu.MemorySpace.VMEM)
    hb = pl.BlockSpec(memory_space=pltpu.MemorySpace.HBM)
    q = pl.pallas_call(
        _mega,
        out_shape=jax.ShapeDtypeStruct((1, 18), F32),
        in_specs=([vm] * 11 + [hb, vm] * 10 + [vm] * 7
                  + [hb, vm, hb, vm, hb, vm, vm]),
        scratch_shapes=[
            pltpu.VMEM((9, 64, 128), F32),
            pltpu.VMEM((9, 128, 128), F32),
            pltpu.VMEM((9, 128, 128), F32),
            pltpu.VMEM((9, 128, 128), F32),
            pltpu.VMEM((9, 128, 128), F32),
            pltpu.VMEM((9, 128, 128), F32),
            pltpu.VMEM((9, 128, 128), F32),
            pltpu.VMEM((9, 128, 128), F32),
            pltpu.VMEM((9, 128, 128), F32),
            pltpu.VMEM((9, 128, 128), F32),
            pltpu.VMEM((8, 128, 512), F32),
            pltpu.VMEM((8, 512, 128), F32),
            pltpu.VMEM((18, 128, 128), F32),
        ] + [pltpu.SemaphoreType.DMA] * 13,
    )(z0, *stage_args(s0), *stage_args(s1), *stage_args(s2),
      _MV0, _MV1, _MV2, _MV3, _SEL, phi2, _RM,
      p['W1'], p['b1'].reshape(8, 1, 512),
      p['W2'], p['b2'].reshape(8, 1, 128),
      whe, _EYE18, p['b_head'].reshape(1, 18))
    return q.reshape(18)
